# baseline (speedup 1.0000x reference)
import jax
import jax.numpy as jnp
from jax import lax
from jax.experimental import pallas as pl
from jax.experimental.pallas import tpu as pltpu

B_ = 64
NP_ = 576
PD_ = 768
CB_ = 16
NCH_ = B_ // CB_
K_ = 2
LEAD_ = 1


def _pipe_kernel(x_hbm, t_hbm, o_hbm, tbuf, ring, isem, osem, tsem):
    tcopy = pltpu.make_async_copy(t_hbm, tbuf, tsem)
    tcopy.start()
    tcopy.wait()

    def in_copy(c, j):
        return pltpu.make_async_copy(
            x_hbm.at[pl.ds(c * CB_, CB_)],
            ring.at[pl.ds(j * CB_, CB_)],
            isem.at[j],
        )

    def out_copy(c, j):
        return pltpu.make_async_copy(
            ring.at[pl.ds(j * CB_, CB_)],
            o_hbm.at[pl.ds(c * CB_, CB_)],
            osem.at[j],
        )

    for c in range(LEAD_):
        in_copy(c, c % K_).start()

    def step(c, carry):
        @pl.when(c >= K_ - LEAD_)
        def _():
            cd = lax.max(c - (K_ - LEAD_), 0)
            out_copy(cd, lax.rem(cd, K_)).wait()

        @pl.when(c + LEAD_ < NCH_)
        def _():
            cn = c + LEAD_
            in_copy(cn, lax.rem(cn, K_)).start()

        j = lax.rem(c, K_)
        in_copy(c, j).wait()
        sl = pl.ds(j * CB_, CB_)
        ring[sl] = ring[sl] + tbuf[...]
        out_copy(c, j).start()
        return carry

    lax.fori_loop(0, NCH_, step, 0)

    for c in range(NCH_ - (K_ - LEAD_), NCH_):
        out_copy(c, c % K_).wait()


def kernel(encoded_patches, pos_table):
    return pl.pallas_call(
        _pipe_kernel,
        in_specs=[
            pl.BlockSpec(memory_space=pltpu.HBM),
            pl.BlockSpec(memory_space=pltpu.HBM),
        ],
        out_specs=pl.BlockSpec(memory_space=pltpu.HBM),
        out_shape=jax.ShapeDtypeStruct(encoded_patches.shape, encoded_patches.dtype),
        scratch_shapes=[
            pltpu.VMEM((NP_, PD_), jnp.float32),
            pltpu.VMEM((K_ * CB_, NP_, PD_), jnp.float32),
            pltpu.SemaphoreType.DMA((K_,)),
            pltpu.SemaphoreType.DMA((K_,)),
            pltpu.SemaphoreType.DMA,
        ],
    )(encoded_patches, pos_table)


# speedup vs baseline: 1.0319x; 1.0319x over previous
import jax
import jax.numpy as jnp
from jax.experimental import pallas as pl

NP_ = 576
PD_ = 768


def _add_kernel(x_ref, t_ref, o_ref):
    o_ref[...] = x_ref[...] + t_ref[...]


BB_ = 8


def kernel(encoded_patches, pos_table):
    b = encoded_patches.shape[0]
    return pl.pallas_call(
        _add_kernel,
        grid=(b // BB_,),
        in_specs=[
            pl.BlockSpec((BB_, NP_, PD_), lambda i: (i, 0, 0)),
            pl.BlockSpec((NP_, PD_), lambda i: (0, 0)),
        ],
        out_specs=pl.BlockSpec((BB_, NP_, PD_), lambda i: (i, 0, 0)),
        out_shape=jax.ShapeDtypeStruct(encoded_patches.shape, encoded_patches.dtype),
    )(encoded_patches, pos_table)
